# Initial kernel scaffold; baseline (speedup 1.0000x reference)
import jax
import jax.numpy as jnp
from jax.experimental import pallas as pl


def kernel(x, emb_table):
    raise NotImplementedError("write your pallas kernel here")


# speedup vs baseline: 1.7254x; 1.7254x over previous
import jax
import jax.numpy as jnp
from jax.experimental import pallas as pl


_BT = 512


def _add_kernel(x_ref, emb_ref, o_ref):
    o_ref[...] = x_ref[...] + emb_ref[...][None, :, :]


def kernel(x, emb_table):
    B, T, D = x.shape
    grid = (T // _BT,)
    return pl.pallas_call(
        _add_kernel,
        grid=grid,
        in_specs=[
            pl.BlockSpec((B, _BT, D), lambda i: (0, i, 0)),
            pl.BlockSpec((_BT, D), lambda i: (i, 0)),
        ],
        out_specs=pl.BlockSpec((B, _BT, D), lambda i: (0, i, 0)),
        out_shape=jax.ShapeDtypeStruct((B, T, D), x.dtype),
    )(x, emb_table)
